# baseline (speedup 1.0000x reference)
import jax
import jax.numpy as jnp
from jax import lax
from jax.experimental import pallas as pl
from jax.experimental.pallas import tpu as pltpu
from jax.experimental.pallas import tpu_sc as plsc

NUM_FIELDS = 26
VOCAB = 100000
EMBED_DIM = 32
BATCH = 16384

NUM_CORES = 2
NUM_SUBCORES = 16
NUM_WORKERS = NUM_CORES * NUM_SUBCORES
BPW = BATCH // NUM_WORKERS
CHUNK = 128
NCHUNK = BPW // CHUNK
PAD_DIM = 128
NSTEP = NUM_FIELDS * NCHUNK
NBUF = 6
LOOKAHEAD = 3


def _body(inst_hbm, w_hbm, out_hbm, gidx_v, buf_v, *sems):
    gsems = sems[:NBUF]
    wsems = sems[NBUF:]
    wid = lax.axis_index("s") * NUM_CORES + lax.axis_index("c")
    base = wid * BPW
    for c in range(NCHUNK):
        pltpu.sync_copy(inst_hbm.at[pl.ds(base + c * CHUNK, CHUNK)],
                        gidx_v.at[c])

    def gather_pair(st, s):
        f, c = st // NCHUNK, st % NCHUNK
        return (
            w_hbm.at[pl.ds(f * VOCAB, VOCAB)].at[gidx_v.at[c]],
            buf_v.at[s],
        )

    def wb_pair(st, s):
        f, c = st // NCHUNK, st % NCHUNK
        return (
            buf_v.at[s, :, pl.ds(0, EMBED_DIM)],
            out_hbm.at[pl.ds(base + c * CHUNK, CHUNK),
                       pl.ds(f * EMBED_DIM, EMBED_DIM)],
        )

    def issue_gather(st, s):
        src, dst = gather_pair(st, s)
        pltpu.async_copy(src, dst, gsems[s])

    def wait_gather(st, s):
        src, dst = gather_pair(st, s)
        pltpu.make_async_copy(src, dst, gsems[s]).wait()

    def issue_wb(st, s):
        src, dst = wb_pair(st, s)
        pltpu.async_copy(src, dst, wsems[s])

    def wait_wb(st, s):
        src, dst = wb_pair(st, s)
        pltpu.make_async_copy(src, dst, wsems[s]).wait()

    for st in range(min(LOOKAHEAD, NSTEP)):
        issue_gather(st, st % NBUF)
    for st in range(NSTEP):
        s = st % NBUF
        nst = st + LOOKAHEAD
        if nst < NSTEP:
            ps = nst % NBUF
            if nst - NBUF >= 0:
                wait_wb(nst - NBUF, ps)
            issue_gather(nst, ps)
        wait_gather(st, s)
        issue_wb(st, s)
    for st in range(max(0, NSTEP - NBUF), NSTEP):
        wait_wb(st, st % NBUF)


PAD_BM = 5000
PAD_NB = VOCAB // PAD_BM


def _pad_body(w_ref, o_ref):
    o_ref[:, :EMBED_DIM] = w_ref[0]


def kernel(instance, W):
    idx = instance.astype(jnp.int32)
    w_flat = pl.pallas_call(
        _pad_body,
        grid=(NUM_FIELDS, PAD_NB),
        in_specs=[pl.BlockSpec((1, PAD_BM, EMBED_DIM), lambda f, i: (f, i, 0))],
        out_specs=pl.BlockSpec((PAD_BM, PAD_DIM), lambda f, i: (f * PAD_NB + i, 0)),
        out_shape=jax.ShapeDtypeStruct((NUM_FIELDS * VOCAB, PAD_DIM), jnp.float32),
    )(W)
    mesh = plsc.VectorSubcoreMesh(core_axis_name="c", subcore_axis_name="s")
    out = pl.kernel(
        _body,
        out_type=jax.ShapeDtypeStruct((BATCH, NUM_FIELDS * EMBED_DIM), jnp.float32),
        mesh=mesh,
        scratch_types=[
            pltpu.VMEM((NCHUNK, CHUNK), jnp.int32),
            pltpu.VMEM((NBUF, CHUNK, PAD_DIM), jnp.float32),
        ] + [pltpu.SemaphoreType.DMA] * (2 * NBUF),
        compiler_params=pltpu.CompilerParams(use_tc_tiling_on_sc=False),
    )(idx, w_flat)
    return out


# speedup vs baseline: 1.3900x; 1.3900x over previous
import jax
import jax.numpy as jnp
from jax import lax
from jax.experimental import pallas as pl
from jax.experimental.pallas import tpu as pltpu
from jax.experimental.pallas import tpu_sc as plsc

NUM_FIELDS = 26
VOCAB = 100000
EMBED_DIM = 32
BATCH = 16384

NUM_CORES = 2
NUM_SUBCORES = 16
NUM_WORKERS = NUM_CORES * NUM_SUBCORES
BPW = BATCH // NUM_WORKERS
CHUNK = 128
NCHUNK = BPW // CHUNK
NBUF = 6
LOOKAHEAD = 4


def _body(inst_hbm, w_hbm, out_hbm, gidx_v, buf_v, *sems):
    gsems = sems[:NBUF]
    wsems = sems[NBUF:]
    wid = lax.axis_index("s") * NUM_CORES + lax.axis_index("c")
    base = wid * BPW
    for c in range(NCHUNK):
        pltpu.sync_copy(inst_hbm.at[pl.ds(base + c * CHUNK, CHUNK)],
                        gidx_v.at[c])

    def gather_pairs(f, s):
        return [(
            w_hbm.at[f].at[gidx_v.at[c]],
            buf_v.at[s, pl.ds(c * CHUNK, CHUNK), :],
        ) for c in range(NCHUNK)]

    def wb_pair(f, s):
        return (
            buf_v.at[s],
            out_hbm.at[pl.ds(base, BPW), pl.ds(f * EMBED_DIM, EMBED_DIM)],
        )

    def issue_gathers(f, s):
        for src, dst in gather_pairs(f, s):
            pltpu.async_copy(src, dst, gsems[s])

    def wait_gathers(f, s):
        for src, dst in gather_pairs(f, s):
            pltpu.make_async_copy(src, dst, gsems[s]).wait()

    def issue_wb(f, s):
        src, dst = wb_pair(f, s)
        pltpu.async_copy(src, dst, wsems[s])

    def wait_wb(f, s):
        src, dst = wb_pair(f, s)
        pltpu.make_async_copy(src, dst, wsems[s]).wait()

    for f in range(min(LOOKAHEAD, NUM_FIELDS)):
        issue_gathers(f, f % NBUF)
    for f in range(NUM_FIELDS):
        s = f % NBUF
        nf = f + LOOKAHEAD
        if nf < NUM_FIELDS:
            ps = nf % NBUF
            if nf - NBUF >= 0:
                wait_wb(nf - NBUF, ps)
            issue_gathers(nf, ps)
        wait_gathers(f, s)
        issue_wb(f, s)
    for f in range(max(0, NUM_FIELDS - NBUF), NUM_FIELDS):
        wait_wb(f, f % NBUF)


def kernel(instance, W):
    idx = instance.astype(jnp.int32)
    mesh = plsc.VectorSubcoreMesh(core_axis_name="c", subcore_axis_name="s")
    out = pl.kernel(
        _body,
        out_type=jax.ShapeDtypeStruct((BATCH, NUM_FIELDS * EMBED_DIM), jnp.float32),
        mesh=mesh,
        scratch_types=[
            pltpu.VMEM((NCHUNK, CHUNK), jnp.int32),
            pltpu.VMEM((NBUF, BPW, EMBED_DIM), jnp.float32),
        ] + [pltpu.SemaphoreType.DMA] * (2 * NBUF),
        compiler_params=pltpu.CompilerParams(use_tc_tiling_on_sc=False),
    )(idx, W)
    return out
